# baseline (speedup 1.0000x reference)
import functools

import jax
import jax.numpy as jnp
from jax import lax
from jax.experimental import pallas as pl
from jax.experimental.pallas import tpu as pltpu
from jax.experimental.pallas import tpu_sc as plsc

_NUM_CORES = 2
_NUM_SUBCORES = 16
_NUM_WORKERS = _NUM_CORES * _NUM_SUBCORES
_CHUNK_ROWS = 8
_NUM_BUFS = 14


def _make_copy(seq_len: int, d: int, dtype):
    rows_per_w = seq_len // _NUM_WORKERS
    n_chunks = rows_per_w // _CHUNK_ROWS
    n_bufs = min(_NUM_BUFS, n_chunks)
    mesh = plsc.VectorSubcoreMesh(
        core_axis_name="c",
        subcore_axis_name="s",
        num_cores=_NUM_CORES,
        num_subcores=_NUM_SUBCORES,
    )

    @functools.partial(
        pl.kernel,
        out_type=jax.ShapeDtypeStruct((seq_len, d), dtype),
        mesh=mesh,
        scratch_types=(
            [pltpu.VMEM((_CHUNK_ROWS, d), dtype)] * n_bufs
            + [pltpu.SemaphoreType.DMA] * (2 * n_bufs)
        ),
    )
    def copy_rows(table_hbm, out_hbm, *rest):
        bufs = rest[:n_bufs]
        gsems = rest[n_bufs : 2 * n_bufs]
        ssems = rest[2 * n_bufs :]
        wid = lax.axis_index("s") * _NUM_CORES + lax.axis_index("c")
        base = wid * rows_per_w

        def row_slice(ref, j):
            return ref.at[pl.ds(base + j * _CHUNK_ROWS, _CHUNK_ROWS)]

        gathers = []
        for j in range(n_bufs):
            g = pltpu.make_async_copy(row_slice(table_hbm, j), bufs[j], gsems[j])
            g.start()
            gathers.append(g)
        scatters = []
        for j in range(n_chunks):
            b = j % n_bufs
            if j >= n_bufs:
                scatters[j - n_bufs].wait()
                g = pltpu.make_async_copy(row_slice(table_hbm, j), bufs[b], gsems[b])
                g.start()
                gathers.append(g)
            gathers[j].wait()
            sc = pltpu.make_async_copy(bufs[b], row_slice(out_hbm, j), ssems[b])
            sc.start()
            scatters.append(sc)
        for j in range(max(0, n_chunks - n_bufs), n_chunks):
            scatters[j].wait()

    return copy_rows


def kernel(x, pos_emb):
    seq_len = x.shape[1]
    d = pos_emb.shape[1]
    return _make_copy(seq_len, d, pos_emb.dtype)(pos_emb)


# speedup vs baseline: 1.0366x; 1.0366x over previous
import functools

import jax
import jax.numpy as jnp
from jax import lax
from jax.experimental import pallas as pl
from jax.experimental.pallas import tpu as pltpu
from jax.experimental.pallas import tpu_sc as plsc

_NUM_CORES = 2
_NUM_SUBCORES = 16
_NUM_WORKERS = _NUM_CORES * _NUM_SUBCORES
_TILE_CHUNK = 16
_SPMEM_CHUNK = 32
_SPMEM_CHUNKS = 2
_SPMEM_ROWS = _SPMEM_CHUNK * _SPMEM_CHUNKS


def _make_copy(seq_len: int, d: int, dtype):
    rows_per_w = seq_len // _NUM_WORKERS
    tile_rows = rows_per_w - _SPMEM_ROWS
    n_tile_chunks = tile_rows // _TILE_CHUNK
    mesh = plsc.VectorSubcoreMesh(
        core_axis_name="c",
        subcore_axis_name="s",
        num_cores=_NUM_CORES,
        num_subcores=_NUM_SUBCORES,
    )

    @functools.partial(
        pl.kernel,
        out_type=jax.ShapeDtypeStruct((seq_len, d), dtype),
        mesh=mesh,
        scratch_types=(
            [pltpu.VMEM((_TILE_CHUNK, d), dtype)] * n_tile_chunks
            + [pltpu.VMEM_SHARED((_NUM_SUBCORES * _SPMEM_ROWS, d), dtype)]
            + [pltpu.SemaphoreType.DMA] * (2 * n_tile_chunks + 2 * _SPMEM_CHUNKS)
        ),
    )
    def copy_rows(table_hbm, out_hbm, *rest):
        bufs = rest[:n_tile_chunks]
        spmem = rest[n_tile_chunks]
        sems = rest[n_tile_chunks + 1 :]
        tg = sems[:n_tile_chunks]
        ts = sems[n_tile_chunks : 2 * n_tile_chunks]
        pg = sems[2 * n_tile_chunks : 2 * n_tile_chunks + _SPMEM_CHUNKS]
        ps = sems[2 * n_tile_chunks + _SPMEM_CHUNKS :]
        wid = lax.axis_index("s") * _NUM_CORES + lax.axis_index("c")
        sid = lax.axis_index("s")
        base = wid * rows_per_w

        def hbm_rows(ref, start, n):
            return ref.at[pl.ds(start, n)]

        tile_gathers = []
        for j in range(n_tile_chunks):
            g = pltpu.make_async_copy(
                hbm_rows(table_hbm, base + j * _TILE_CHUNK, _TILE_CHUNK),
                bufs[j],
                tg[j],
            )
            g.start()
            tile_gathers.append(g)
        sp_base = base + tile_rows
        sp_slots = []
        sp_gathers = []
        for j in range(_SPMEM_CHUNKS):
            slot = spmem.at[
                pl.ds(sid * _SPMEM_ROWS + j * _SPMEM_CHUNK, _SPMEM_CHUNK)
            ]
            g = pltpu.make_async_copy(
                hbm_rows(table_hbm, sp_base + j * _SPMEM_CHUNK, _SPMEM_CHUNK),
                slot,
                pg[j],
            )
            g.start()
            sp_slots.append(slot)
            sp_gathers.append(g)

        scatters = []
        for j in range(n_tile_chunks):
            tile_gathers[j].wait()
            sc = pltpu.make_async_copy(
                bufs[j], hbm_rows(out_hbm, base + j * _TILE_CHUNK, _TILE_CHUNK), ts[j]
            )
            sc.start()
            scatters.append(sc)
        for j in range(_SPMEM_CHUNKS):
            sp_gathers[j].wait()
            sc = pltpu.make_async_copy(
                sp_slots[j],
                hbm_rows(out_hbm, sp_base + j * _SPMEM_CHUNK, _SPMEM_CHUNK),
                ps[j],
            )
            sc.start()
            scatters.append(sc)
        for sc in scatters:
            sc.wait()

    return copy_rows


def kernel(x, pos_emb):
    seq_len = x.shape[1]
    d = pos_emb.shape[1]
    return _make_copy(seq_len, d, pos_emb.dtype)(pos_emb)


# speedup vs baseline: 1.0384x; 1.0017x over previous
import functools

import jax
import jax.numpy as jnp
from jax import lax
from jax.experimental import pallas as pl
from jax.experimental.pallas import tpu as pltpu
from jax.experimental.pallas import tpu_sc as plsc

_NUM_CORES = 2
_NUM_SUBCORES = 16
_NUM_WORKERS = _NUM_CORES * _NUM_SUBCORES
_CHUNK_ROWS = 16
_NUM_BUFS = 7


def _make_copy(seq_len: int, d: int, dtype):
    rows_per_w = seq_len // _NUM_WORKERS
    n_chunks = rows_per_w // _CHUNK_ROWS
    n_bufs = min(_NUM_BUFS, n_chunks)
    mesh = plsc.VectorSubcoreMesh(
        core_axis_name="c",
        subcore_axis_name="s",
        num_cores=_NUM_CORES,
        num_subcores=_NUM_SUBCORES,
    )

    @functools.partial(
        pl.kernel,
        out_type=jax.ShapeDtypeStruct((seq_len, d), dtype),
        mesh=mesh,
        scratch_types=(
            [pltpu.VMEM((_CHUNK_ROWS, d), dtype)] * n_bufs
            + [pltpu.SemaphoreType.DMA] * (2 * n_bufs)
        ),
    )
    def copy_rows(table_hbm, out_hbm, *rest):
        bufs = rest[:n_bufs]
        gsems = rest[n_bufs : 2 * n_bufs]
        ssems = rest[2 * n_bufs :]
        wid = lax.axis_index("s") * _NUM_CORES + lax.axis_index("c")
        base = wid * rows_per_w

        def row_slice(ref, j):
            return ref.at[pl.ds(base + j * _CHUNK_ROWS, _CHUNK_ROWS)]

        gathers = []
        for j in range(n_bufs):
            g = pltpu.make_async_copy(row_slice(table_hbm, j), bufs[j], gsems[j])
            g.start()
            gathers.append(g)
        scatters = []
        for j in range(n_chunks):
            b = j % n_bufs
            if j >= n_bufs:
                scatters[j - n_bufs].wait()
                g = pltpu.make_async_copy(row_slice(table_hbm, j), bufs[b], gsems[b])
                g.start()
                gathers.append(g)
            gathers[j].wait()
            sc = pltpu.make_async_copy(bufs[b], row_slice(out_hbm, j), ssems[b])
            sc.start()
            scatters.append(sc)
        for j in range(max(0, n_chunks - n_bufs), n_chunks):
            scatters[j].wait()

    return copy_rows


def kernel(x, pos_emb):
    seq_len = x.shape[1]
    d = pos_emb.shape[1]
    return _make_copy(seq_len, d, pos_emb.dtype)(pos_emb)
